# baseline (speedup 1.0000x reference)
import functools

import jax
import jax.numpy as jnp
from jax import lax
from jax.experimental import pallas as pl
from jax.experimental.pallas import tpu as pltpu
from jax.experimental.pallas import tpu_sc as plsc

_TM = 128
_TF = 1024
_TM_LOG2 = 7
_NW = 32
_L = 16


def _sc_route_scatter(te2d, xf, E, K, P, Tpad):
    NC, D = 2, xf.shape[1]
    n_chunks = te2d.shape[0]
    N = n_chunks * _L
    chunks_w = n_chunks // _NW
    rep_w = chunks_w * _L
    tok_w = rep_w // K
    mesh = plsc.VectorSubcoreMesh(core_axis_name="c", subcore_axis_name="s")

    @functools.partial(
        pl.kernel,
        out_type=(
            jax.ShapeDtypeStruct((n_chunks, _L), jnp.int32),
            jax.ShapeDtypeStruct((Tpad,), jnp.int32),
            jax.ShapeDtypeStruct((P, D), jnp.float32),
        ),
        mesh=mesh,
        compiler_params=pltpu.CompilerParams(needs_layout_passes=False),
        scratch_types=[
            pltpu.VMEM((n_chunks, _L), jnp.int32),
            pltpu.VMEM((chunks_w, _L), jnp.int32),
            pltpu.VMEM((Tpad,), jnp.int32),
            pltpu.VMEM((_L, D), jnp.float32),
            pltpu.SemaphoreType.DMA,
        ],
    )
    def k(te_hbm, xf_hbm, pos_hbm, texp_hbm, xg_hbm, te_v, pos_v, texp_v,
          xbuf, sem):
        w = lax.axis_index("s") * NC + lax.axis_index("c")
        my_c0 = w * chunks_w
        pltpu.sync_copy(te_hbm, te_v)
        iota = lax.iota(jnp.int32, _L)

        def splat_i(s):
            return jnp.full((_L,), s, jnp.int32)

        evec = [splat_i(e) for e in range(E)]
        zero = jnp.zeros((_L,), jnp.int32)
        one = splat_i(1)

        def ind(mask):
            return jnp.where(mask, one, zero)

        def count_body(c, accs):
            v = te_v[c]
            inrange = ind(splat_i(c) < splat_i(my_c0))
            new = []
            for e in range(E):
                m = ind(v == evec[e])
                new.append((accs[2 * e] + m, accs[2 * e + 1] + m * inrange))
            return tuple(x for pair in new for x in pair)

        accs = lax.fori_loop(0, n_chunks, count_body, (zero,) * (2 * E))
        tot = zero
        prev = zero
        for e in range(E):
            lane = ind(iota == evec[e])
            tot = tot + lane * splat_i(jnp.sum(accs[2 * e]))
            prev = prev + lane * splat_i(jnp.sum(accs[2 * e + 1]))
        pc = ((tot + splat_i(_TM - 1)) >> splat_i(_TM_LOG2)) << splat_i(_TM_LOG2)
        cpc = jnp.cumsum(pc)
        start = (cpc - pc) + prev

        @pl.when(w == 0)
        def _texp():
            for c in range(Tpad // _L):
                tid = (iota + splat_i(c * _L)) * splat_i(_TM)
                acc = jnp.zeros((_L,), jnp.int32)
                for e in range(E):
                    s_e = jnp.sum(cpc * ind(iota == evec[e]))
                    acc = acc + ind(tid >= splat_i(s_e))
                texp_v[pl.ds(c * _L, _L)] = jnp.minimum(acc, splat_i(E - 1))
            pltpu.sync_copy(texp_v, texp_hbm)

        r = [jnp.sum(start * ind(iota == evec[e])) for e in range(E)]
        for c in range(chunks_w):
            v = te_v[my_c0 + c]
            slot = jnp.zeros((_L,), jnp.int32)
            for e in range(E):
                m = v == evec[e]
                mi = ind(m)
                pref = jnp.cumsum(mi)
                slot = jnp.where(m, splat_i(r[e]) + pref - one, slot)
                r[e] = r[e] + jnp.sum(mi)
            pos_v[c] = slot
        pltpu.sync_copy(pos_v, pos_hbm.at[pl.ds(my_c0, chunks_w)])

        for j in range(tok_w // _L):
            pltpu.sync_copy(xf_hbm.at[pl.ds(w * tok_w + j * _L, _L)], xbuf)
            for kk in range(K):
                rep = iota * splat_i(K) + splat_i(kk + j * _L * K)
                dst = plsc.load_gather(
                    pos_v, [rep >> splat_i(4), rep & splat_i(_L - 1)]
                )
                pltpu.async_copy(xbuf, xg_hbm.at[dst], sem).wait()

    return k


def _sc_combine(h, pos2d, ew2d, Ntok, K):
    NC, D = 2, h.shape[1]
    n_chunks = pos2d.shape[0]
    chunks_w = n_chunks // _NW
    tok_chunk = _L // K
    tok_w = chunks_w * tok_chunk
    mesh = plsc.VectorSubcoreMesh(core_axis_name="c", subcore_axis_name="s")

    @functools.partial(
        pl.kernel,
        out_type=jax.ShapeDtypeStruct((Ntok, D), jnp.float32),
        mesh=mesh,
        compiler_params=pltpu.CompilerParams(needs_layout_passes=False),
        scratch_types=[
            pltpu.VMEM((chunks_w, _L), jnp.int32),
            pltpu.VMEM((chunks_w, _L), jnp.float32),
            pltpu.VMEM((_L, D), jnp.float32),
            pltpu.VMEM((tok_chunk, D), jnp.float32),
            pltpu.SemaphoreType.DMA,
        ],
    )
    def k(h_hbm, pos_hbm, ew_hbm, out_hbm, pos_v, ew_v, hbuf, obuf, sem):
        w = lax.axis_index("s") * NC + lax.axis_index("c")
        my_c0 = w * chunks_w
        pltpu.sync_copy(pos_hbm.at[pl.ds(my_c0, chunks_w)], pos_v)
        pltpu.sync_copy(ew_hbm.at[pl.ds(my_c0, chunks_w)], ew_v)
        iota = lax.iota(jnp.int32, _L)
        fone = jnp.full((_L,), 1.0, jnp.float32)
        fzero = jnp.zeros((_L,), jnp.float32)
        for j in range(chunks_w):
            idx = pos_v[j]
            pltpu.async_copy(h_hbm.at[idx], hbuf, sem).wait()
            sc = ew_v[j]
            sv = [
                jnp.full(
                    (_L,),
                    jnp.sum(
                        sc
                        * jnp.where(
                            iota == jnp.full((_L,), i, jnp.int32), fone, fzero
                        )
                    ),
                    jnp.float32,
                )
                for i in range(_L)
            ]

            def col_body(d, carry):
                col = pl.ds(d * _L, _L)
                for t in range(tok_chunk):
                    acc = hbuf[K * t, col] * sv[K * t]
                    for kk in range(1, K):
                        acc = acc + hbuf[K * t + kk, col] * sv[K * t + kk]
                    obuf[t, col] = acc
                return carry

            lax.fori_loop(0, D // _L, col_body, 0)
            pltpu.sync_copy(
                obuf, out_hbm.at[pl.ds(w * tok_w + j * tok_chunk, tok_chunk)]
            )

    return k


def _l1_block(te_ref, x_ref, w1_ref, h_ref):
    h_ref[...] = jax.nn.gelu(
        jnp.dot(x_ref[...], w1_ref[0], preferred_element_type=jnp.float32)
    )


def _l2_block(te_ref, h_ref, w2_ref, o_ref):
    k = pl.program_id(0)
    t = pl.program_id(1)
    contrib = jnp.dot(h_ref[...], w2_ref[0], preferred_element_type=jnp.float32)
    rows = pl.ds(t * _TM, _TM)

    @pl.when(k == 0)
    def _init():
        o_ref[rows, :] = contrib

    @pl.when(k != 0)
    def _acc():
        o_ref[rows, :] += contrib


def _grouped_mlp(tile_expert, xg, w1, w2):
    P, D = xg.shape
    E, _, FF = w1.shape
    T = P // _TM
    F = FF // _TF
    hid = pl.pallas_call(
        _l1_block,
        grid_spec=pltpu.PrefetchScalarGridSpec(
            num_scalar_prefetch=1,
            grid=(F, T),
            in_specs=[
                pl.BlockSpec((_TM, D), lambda f, t, te: (t, 0)),
                pl.BlockSpec((1, D, _TF), lambda f, t, te: (te[t], 0, f)),
            ],
            out_specs=pl.BlockSpec((_TM, _TF), lambda f, t, te: (t, f)),
        ),
        out_shape=jax.ShapeDtypeStruct((P, FF), jnp.float32),
        compiler_params=pltpu.CompilerParams(
            dimension_semantics=("arbitrary", "arbitrary"),
        ),
    )(tile_expert, xg, w1)
    return pl.pallas_call(
        _l2_block,
        grid_spec=pltpu.PrefetchScalarGridSpec(
            num_scalar_prefetch=1,
            grid=(F, T),
            in_specs=[
                pl.BlockSpec((_TM, _TF), lambda k, t, te: (t, k)),
                pl.BlockSpec((1, _TF, D), lambda k, t, te: (te[t], k, 0)),
            ],
            out_specs=pl.BlockSpec((P, D), lambda k, t, te: (0, 0)),
        ),
        out_shape=jax.ShapeDtypeStruct((P, D), jnp.float32),
        compiler_params=pltpu.CompilerParams(
            dimension_semantics=("arbitrary", "arbitrary"),
        ),
    )(tile_expert, hid, w2)


def kernel(x, scores, expert_weights, top_experts, w1, w2):
    in_shape = x.shape
    D = x.shape[-1]
    E = w1.shape[0]
    K = top_experts.shape[-1]

    xf = x.reshape(-1, D)
    te = top_experts.reshape(-1).astype(jnp.int32)
    N = te.shape[0]
    Ntok = N // K
    P = N + E * _TM
    T = P // _TM
    Tpad = ((T + _L - 1) // _L) * _L

    te2d = te.reshape(-1, _L)
    ew2d = expert_weights.reshape(-1).reshape(-1, _L)

    route = _sc_route_scatter(te2d, xf, E, K, P, Tpad)
    pos2d, texp, xg = route(te2d, xf)
    h = _grouped_mlp(texp[:T], xg, w1, w2)
    combine = _sc_combine(h, pos2d, ew2d, Ntok, K)
    out = combine(h, pos2d, ew2d)
    return out.reshape(in_shape)


# speedup vs baseline: 1.4827x; 1.4827x over previous
import functools

import jax
import jax.numpy as jnp
from jax import lax
from jax.experimental import pallas as pl
from jax.experimental.pallas import tpu as pltpu
from jax.experimental.pallas import tpu_sc as plsc

_TM = 128
_TF = 1024
_TM_LOG2 = 7
_NW = 32
_L = 16


def _sc_route_scatter(te2d, xf, E, K, P, Tpad):
    NC, D = 2, xf.shape[1]
    n_chunks = te2d.shape[0]
    N = n_chunks * _L
    chunks_w = n_chunks // _NW
    rep_w = chunks_w * _L
    tok_w = rep_w // K
    mesh = plsc.VectorSubcoreMesh(core_axis_name="c", subcore_axis_name="s")

    @functools.partial(
        pl.kernel,
        out_type=(
            jax.ShapeDtypeStruct((n_chunks, _L), jnp.int32),
            jax.ShapeDtypeStruct((Tpad,), jnp.int32),
            jax.ShapeDtypeStruct((P, D), jnp.float32),
        ),
        mesh=mesh,
        compiler_params=pltpu.CompilerParams(needs_layout_passes=False),
        scratch_types=[
            pltpu.VMEM((n_chunks, _L), jnp.int32),
            pltpu.VMEM((chunks_w, _L), jnp.int32),
            pltpu.VMEM((Tpad,), jnp.int32),
            pltpu.VMEM((_L, D), jnp.float32),
            pltpu.VMEM((_L, D), jnp.float32),
            pltpu.SemaphoreType.DMA,
            pltpu.SemaphoreType.DMA,
            pltpu.SemaphoreType.DMA,
        ],
    )
    def k(te_hbm, xf_hbm, pos_hbm, texp_hbm, xg_hbm, te_v, pos_v, texp_v,
          xbuf, xbuf2, sem, sem2, ssem):
        w = lax.axis_index("s") * NC + lax.axis_index("c")
        my_c0 = w * chunks_w
        pltpu.sync_copy(te_hbm, te_v)
        iota = lax.iota(jnp.int32, _L)

        def splat_i(s):
            return jnp.full((_L,), s, jnp.int32)

        evec = [splat_i(e) for e in range(E)]
        zero = jnp.zeros((_L,), jnp.int32)
        one = splat_i(1)

        def ind(mask):
            return jnp.where(mask, one, zero)

        def count_body(c, accs):
            v = te_v[c]
            inrange = ind(splat_i(c) < splat_i(my_c0))
            new = []
            for e in range(E):
                m = ind(v == evec[e])
                new.append((accs[2 * e] + m, accs[2 * e + 1] + m * inrange))
            return tuple(x for pair in new for x in pair)

        accs = lax.fori_loop(0, n_chunks, count_body, (zero,) * (2 * E))
        tot = zero
        prev = zero
        for e in range(E):
            lane = ind(iota == evec[e])
            tot = tot + lane * splat_i(jnp.sum(accs[2 * e]))
            prev = prev + lane * splat_i(jnp.sum(accs[2 * e + 1]))
        pc = ((tot + splat_i(_TM - 1)) >> splat_i(_TM_LOG2)) << splat_i(_TM_LOG2)
        cpc = jnp.cumsum(pc)
        start = (cpc - pc) + prev

        @pl.when(w == 0)
        def _texp():
            for c in range(Tpad // _L):
                tid = (iota + splat_i(c * _L)) * splat_i(_TM)
                acc = jnp.zeros((_L,), jnp.int32)
                for e in range(E):
                    s_e = jnp.sum(cpc * ind(iota == evec[e]))
                    acc = acc + ind(tid >= splat_i(s_e))
                texp_v[pl.ds(c * _L, _L)] = jnp.minimum(acc, splat_i(E - 1))
            pltpu.sync_copy(texp_v, texp_hbm)

        r = [jnp.sum(start * ind(iota == evec[e])) for e in range(E)]
        for c in range(chunks_w):
            v = te_v[my_c0 + c]
            slot = jnp.zeros((_L,), jnp.int32)
            for e in range(E):
                m = v == evec[e]
                mi = ind(m)
                pref = jnp.cumsum(mi)
                slot = jnp.where(m, splat_i(r[e]) + pref - one, slot)
                r[e] = r[e] + jnp.sum(mi)
            pos_v[c] = slot
        pltpu.sync_copy(pos_v, pos_hbm.at[pl.ds(my_c0, chunks_w)])

        nj = tok_w // _L
        xbufs = (xbuf, xbuf2)
        lsems = (sem, sem2)

        def load(j):
            return pltpu.async_copy(
                xf_hbm.at[pl.ds(w * tok_w + j * _L, _L)], xbufs[j & 1],
                lsems[j & 1],
            )

        pending = load(0)
        for j in range(nj):
            nxt = load(j + 1) if j + 1 < nj else None
            pending.wait()
            scats = []
            for kk in range(K):
                rep = iota * splat_i(K) + splat_i(kk + j * _L * K)
                dst = plsc.load_gather(
                    pos_v, [rep >> splat_i(4), rep & splat_i(_L - 1)]
                )
                scats.append(
                    pltpu.async_copy(xbufs[j & 1], xg_hbm.at[dst], ssem)
                )
            for s in scats:
                s.wait()
            pending = nxt

    return k


def _sc_combine(h, pos2d, ew2d, Ntok, K):
    NC, D = 2, h.shape[1]
    n_chunks = pos2d.shape[0]
    chunks_w = n_chunks // _NW
    tok_chunk = _L // K
    tok_w = chunks_w * tok_chunk
    mesh = plsc.VectorSubcoreMesh(core_axis_name="c", subcore_axis_name="s")

    @functools.partial(
        pl.kernel,
        out_type=jax.ShapeDtypeStruct((Ntok, D), jnp.float32),
        mesh=mesh,
        compiler_params=pltpu.CompilerParams(needs_layout_passes=False),
        scratch_types=[
            pltpu.VMEM((chunks_w, _L), jnp.int32),
            pltpu.VMEM((chunks_w, _L), jnp.float32),
            pltpu.VMEM((_L, D), jnp.float32),
            pltpu.VMEM((_L, D), jnp.float32),
            pltpu.VMEM((tok_chunk, D), jnp.float32),
            pltpu.SemaphoreType.DMA,
            pltpu.SemaphoreType.DMA,
        ],
    )
    def k(h_hbm, pos_hbm, ew_hbm, out_hbm, pos_v, ew_v, hbuf0, hbuf1, obuf,
          sem0, sem1):
        w = lax.axis_index("s") * NC + lax.axis_index("c")
        my_c0 = w * chunks_w
        pltpu.sync_copy(pos_hbm.at[pl.ds(my_c0, chunks_w)], pos_v)
        pltpu.sync_copy(ew_hbm.at[pl.ds(my_c0, chunks_w)], ew_v)
        iota = lax.iota(jnp.int32, _L)
        fone = jnp.full((_L,), 1.0, jnp.float32)
        fzero = jnp.zeros((_L,), jnp.float32)
        hbufs = (hbuf0, hbuf1)
        sems = (sem0, sem1)

        def gather(j):
            return pltpu.async_copy(
                h_hbm.at[pos_v[j]], hbufs[j & 1], sems[j & 1]
            )

        pending = gather(0)
        for j in range(chunks_w):
            nxt = gather(j + 1) if j + 1 < chunks_w else None
            pending.wait()
            hbuf = hbufs[j & 1]
            sc = ew_v[j]
            sv = [
                jnp.full(
                    (_L,),
                    jnp.sum(
                        sc
                        * jnp.where(
                            iota == jnp.full((_L,), i, jnp.int32), fone, fzero
                        )
                    ),
                    jnp.float32,
                )
                for i in range(_L)
            ]

            def col_body(d, carry):
                col = pl.ds(d * _L, _L)
                for t in range(tok_chunk):
                    acc = hbuf[K * t, col] * sv[K * t]
                    for kk in range(1, K):
                        acc = acc + hbuf[K * t + kk, col] * sv[K * t + kk]
                    obuf[t, col] = acc
                return carry

            lax.fori_loop(0, D // _L, col_body, 0)
            pltpu.sync_copy(
                obuf, out_hbm.at[pl.ds(w * tok_w + j * tok_chunk, tok_chunk)]
            )
            pending = nxt

    return k


def _mlp_block(te_ref, x_ref, w1_ref, w2_ref, o_ref):
    f = pl.program_id(0)
    t = pl.program_id(1)
    hidden = jax.nn.gelu(
        jnp.dot(x_ref[...], w1_ref[0], preferred_element_type=jnp.float32)
    )
    contrib = jnp.dot(hidden, w2_ref[0], preferred_element_type=jnp.float32)
    rows = pl.ds(t * _TM, _TM)

    @pl.when(f == 0)
    def _init():
        o_ref[rows, :] = contrib

    @pl.when(f != 0)
    def _acc():
        o_ref[rows, :] += contrib


def _grouped_mlp(tile_expert, xg, w1, w2):
    P, D = xg.shape
    E, _, FF = w1.shape
    T = P // _TM
    F = FF // _TF
    grid_spec = pltpu.PrefetchScalarGridSpec(
        num_scalar_prefetch=1,
        grid=(F, T),
        in_specs=[
            pl.BlockSpec((_TM, D), lambda f, t, te: (t, 0)),
            pl.BlockSpec((1, D, _TF), lambda f, t, te: (te[t], 0, f)),
            pl.BlockSpec((1, _TF, D), lambda f, t, te: (te[t], f, 0)),
        ],
        out_specs=pl.BlockSpec((P, D), lambda f, t, te: (0, 0)),
    )
    return pl.pallas_call(
        _mlp_block,
        grid_spec=grid_spec,
        out_shape=jax.ShapeDtypeStruct((P, D), jnp.float32),
        compiler_params=pltpu.CompilerParams(
            dimension_semantics=("arbitrary", "arbitrary"),
        ),
    )(tile_expert, xg, w1, w2)


def kernel(x, scores, expert_weights, top_experts, w1, w2):
    in_shape = x.shape
    D = x.shape[-1]
    E = w1.shape[0]
    K = top_experts.shape[-1]

    xf = x.reshape(-1, D)
    te = top_experts.reshape(-1).astype(jnp.int32)
    N = te.shape[0]
    Ntok = N // K
    P = N + E * _TM
    T = P // _TM
    Tpad = ((T + _L - 1) // _L) * _L

    te2d = te.reshape(-1, _L)
    ew2d = expert_weights.reshape(-1).reshape(-1, _L)

    route = _sc_route_scatter(te2d, xf, E, K, P, Tpad)
    pos2d, texp, xg = route(te2d, xf)
    h = _grouped_mlp(texp[:T], xg, w1, w2)
    combine = _sc_combine(h, pos2d, ew2d, Ntok, K)
    out = combine(h, pos2d, ew2d)
    return out.reshape(in_shape)
